# baseline (speedup 1.0000x reference)
import functools
import math

import jax
import jax.numpy as jnp
from jax import lax
from jax.experimental import pallas as pl
from jax.experimental.pallas import tpu as pltpu
from jax.experimental.pallas import tpu_sc as plsc

N = 10000
E = 320000
F_IN = 128
HID = 32
PERIODS = 12

N_PAD = 10240
CT = 6
COLT = 128
DEGW = 128
CHUNK = 128
NSUB = 16
NBUF = 2
NROWS_B = 160
EP_B = NSUB * NROWS_B * CHUNK
NBUF_A = 4
NROWS_A = 80
EP_A = 32 * NROWS_A * CHUNK
SLICE = N_PAD // NSUB

_mesh = plsc.VectorSubcoreMesh(core_axis_name="c", subcore_axis_name="s")


@functools.partial(
    pl.kernel,
    out_type=jax.ShapeDtypeStruct((2 * N_PAD, DEGW), jnp.float32),
    mesh=_mesh,
    scratch_types=[
        pltpu.VMEM((NROWS_A, CHUNK), jnp.int32),
        pltpu.VMEM((CHUNK, DEGW), jnp.float32),
        pltpu.VMEM_SHARED((N_PAD, DEGW), jnp.float32),
    ],
)
def _deg_kernel(didx_hbm, ones_hbm, zeros_hbm, out_hbm, didx_v, ones_v, accum):
    c = lax.axis_index("c")
    s = lax.axis_index("s")
    w = c * NSUB + s
    pltpu.sync_copy(ones_hbm, ones_v)
    pltpu.sync_copy(didx_hbm.at[pl.ds(w * NROWS_A, NROWS_A)], didx_v)
    pltpu.sync_copy(zeros_hbm.at[pl.ds(s * SLICE, SLICE)],
                    accum.at[pl.ds(s * SLICE, SLICE)])
    plsc.subcore_barrier()

    def body(j, carry):
        pltpu.sync_copy(ones_v, accum.at[didx_v.at[j]], add=True)
        return carry

    lax.fori_loop(0, NROWS_A, body, 0)
    plsc.subcore_barrier()
    pltpu.sync_copy(accum.at[pl.ds(s * SLICE, SLICE)],
                    out_hbm.at[pl.ds(c * N_PAD + s * SLICE, SLICE)])


@functools.partial(
    pl.kernel,
    out_type=jax.ShapeDtypeStruct((CT * N_PAD, COLT), jnp.float32),
    mesh=_mesh,
    scratch_types=[
        pltpu.VMEM((CHUNK,), jnp.int32),
        pltpu.VMEM((CHUNK,), jnp.int32),
        pltpu.VMEM_SHARED((N_PAD, COLT), jnp.float32),
        pltpu.VMEM((CHUNK, COLT), jnp.float32),
        pltpu.SemaphoreType.DMA,
    ],
)
def _prop_kernel(sidx_hbm, didx_hbm, ghat_hbm, zeros_hbm, out_hbm,
                 sidx_v, didx_v, accum, rows_v, sem):
    c = lax.axis_index("c")
    s = lax.axis_index("s")
    for k in range(CT // 2):
        t = c * (CT // 2) + k
        pltpu.sync_copy(zeros_hbm.at[pl.ds(s * SLICE, SLICE)],
                        accum.at[pl.ds(s * SLICE, SLICE)])
        plsc.subcore_barrier()

        def body(j, carry):
            row = s * NROWS_B + j
            pltpu.sync_copy(sidx_hbm.at[t * (NSUB * NROWS_B) + row], sidx_v)
            pltpu.sync_copy(didx_hbm.at[row], didx_v)
            pltpu.async_copy(ghat_hbm.at[sidx_v], rows_v, sem).wait()
            pltpu.sync_copy(rows_v, accum.at[didx_v], add=True)
            return carry

        lax.fori_loop(0, NROWS_B, body, 0)
        plsc.subcore_barrier()
        pltpu.sync_copy(accum.at[pl.ds(s * SLICE, SLICE)],
                        out_hbm.at[pl.ds(t * N_PAD + s * SLICE, SLICE)])


_BN = 1280


def _prep_body(x_ref, d0_ref, d1_ref, wz_ref, lzw_ref, wh_ref, lhw_ref, out_ref):
    wzp = jnp.dot(wz_ref[...], lzw_ref[:HID, :], preferred_element_type=jnp.float32)
    whp = jnp.dot(wh_ref[...], lhw_ref[:HID, :], preferred_element_type=jnp.float32)
    wcat = jnp.concatenate([wzp, whp], axis=1)
    deg = d0_ref[:, 0] + d1_ref[:, 0] + 1.0
    dinv = lax.rsqrt(deg)
    for p in range(PERIODS):
        g = jnp.dot(x_ref[p], wcat, preferred_element_type=jnp.float32)
        t, half = p // 2, p % 2
        out_ref[t, :, half * 64:(half + 1) * 64] = g * dinv[:, None]


def _post_body(c0_ref, gh_ref, d0_ref, d1_ref, att_ref, bz_ref, lzw_ref,
               lzb_ref, bh_ref, lhw_ref, lhb_ref, gam_ref, bet_ref,
               wout_ref, bout_ref, out_ref):
    probs = jax.nn.softmax(att_ref[...], axis=-1)
    bzp = jnp.dot(bz_ref[...], lzw_ref[:HID, :],
                  preferred_element_type=jnp.float32) + lzb_ref[...]
    bhp = jnp.dot(bh_ref[...], lhw_ref[:HID, :],
                  preferred_element_type=jnp.float32) + lhb_ref[...]
    deg = d0_ref[:, 0] + d1_ref[:, 0] + 1.0
    dinv = lax.rsqrt(deg)[:, None]
    acc = jnp.zeros((c0_ref.shape[1], HID), jnp.float32)
    for p in range(PERIODS):
        t, half = p // 2, p % 2
        a = half * 64
        cz = (c0_ref[t, :, a:a + HID] + gh_ref[t, :, a:a + HID]) * dinv + bzp
        ch = (c0_ref[t, :, a + HID:a + 64] + gh_ref[t, :, a + HID:a + 64]) * dinv + bhp
        hp = (1.0 - jax.nn.sigmoid(cz)) * jnp.tanh(ch)
        acc = acc + probs[0:1, p:p + 1] * hp
    h = jnp.where(acc > 0, acc, 0.01 * acc)
    h = (h * (1.0 / math.sqrt(1.0 + 1e-5))) * gam_ref[...] + bet_ref[...]
    out_ref[...] = jnp.dot(h, wout_ref[...],
                           preferred_element_type=jnp.float32) + bout_ref[...]


def kernel(x, edge_index, attention, Wz, bz, LzW, Lzb, Wr, br, LrW, Lrb,
           Wh, bh, LhW, Lhb, gamma, beta, Wout, bout):
    del Wr, br, LrW, Lrb
    f32 = jnp.float32
    s = edge_index[0]
    d = edge_index[1]

    d_a = jnp.concatenate([d, jnp.full((EP_A - E,), N, jnp.int32)])
    didx_a = d_a.reshape(32 * NROWS_A, CHUNK)
    s_b = jnp.concatenate([s, jnp.zeros((EP_B - E,), jnp.int32)])
    d_b = jnp.concatenate([d, jnp.full((EP_B - E,), N, jnp.int32)])
    didx_b = d_b.reshape(NSUB * NROWS_B, CHUNK)
    offs = (jnp.arange(CT, dtype=jnp.int32) * N_PAD)[:, None, None]
    sidx_b = (s_b.reshape(NSUB * NROWS_B, CHUNK)[None] + offs
              ).reshape(CT * NSUB * NROWS_B, CHUNK)

    ones128 = jnp.ones((CHUNK, DEGW), f32)
    zeros128 = jnp.zeros((N_PAD, DEGW), f32)

    xT = jnp.pad(x.transpose(2, 0, 1), ((0, 0), (0, N_PAD - N), (0, 0)))

    degp = _deg_kernel(didx_a, ones128, zeros128)
    deg0 = degp[:N_PAD]
    deg1 = degp[N_PAD:]

    nb = N_PAD // _BN
    ghat = pl.pallas_call(
        _prep_body,
        grid=(nb,),
        in_specs=[
            pl.BlockSpec((PERIODS, _BN, F_IN), lambda i: (0, i, 0)),
            pl.BlockSpec((_BN, DEGW), lambda i: (i, 0)),
            pl.BlockSpec((_BN, DEGW), lambda i: (i, 0)),
            pl.BlockSpec((F_IN, HID), lambda i: (0, 0)),
            pl.BlockSpec((2 * HID, HID), lambda i: (0, 0)),
            pl.BlockSpec((F_IN, HID), lambda i: (0, 0)),
            pl.BlockSpec((2 * HID, HID), lambda i: (0, 0)),
        ],
        out_specs=pl.BlockSpec((CT, _BN, COLT), lambda i: (0, i, 0)),
        out_shape=jax.ShapeDtypeStruct((CT, N_PAD, COLT), f32),
    )(xT, deg0, deg1, Wz, LzW, Wh, LhW)

    c0 = _prop_kernel(sidx_b, didx_b, ghat.reshape(CT * N_PAD, COLT), zeros128)
    c0 = c0.reshape(CT, N_PAD, COLT)

    out = pl.pallas_call(
        _post_body,
        grid=(nb,),
        in_specs=[
            pl.BlockSpec((CT, _BN, COLT), lambda i: (0, i, 0)),
            pl.BlockSpec((CT, _BN, COLT), lambda i: (0, i, 0)),
            pl.BlockSpec((_BN, DEGW), lambda i: (i, 0)),
            pl.BlockSpec((_BN, DEGW), lambda i: (i, 0)),
            pl.BlockSpec((1, PERIODS), lambda i: (0, 0)),
            pl.BlockSpec((1, HID), lambda i: (0, 0)),
            pl.BlockSpec((2 * HID, HID), lambda i: (0, 0)),
            pl.BlockSpec((1, HID), lambda i: (0, 0)),
            pl.BlockSpec((1, HID), lambda i: (0, 0)),
            pl.BlockSpec((2 * HID, HID), lambda i: (0, 0)),
            pl.BlockSpec((1, HID), lambda i: (0, 0)),
            pl.BlockSpec((1, HID), lambda i: (0, 0)),
            pl.BlockSpec((1, HID), lambda i: (0, 0)),
            pl.BlockSpec((HID, PERIODS), lambda i: (0, 0)),
            pl.BlockSpec((1, PERIODS), lambda i: (0, 0)),
        ],
        out_specs=pl.BlockSpec((_BN, PERIODS), lambda i: (i, 0)),
        out_shape=jax.ShapeDtypeStruct((N_PAD, PERIODS), f32),
    )(c0, ghat, deg0, deg1, attention[None, :], bz[None, :], LzW,
      Lzb[None, :], bh[None, :], LhW, Lhb[None, :], gamma[None, :],
      beta[None, :], Wout, bout[None, :])
    return out[:N]


# speedup vs baseline: 1.5720x; 1.5720x over previous
import functools
import math

import jax
import jax.numpy as jnp
from jax import lax
from jax.experimental import pallas as pl
from jax.experimental.pallas import tpu as pltpu
from jax.experimental.pallas import tpu_sc as plsc

N = 10000
E = 320000
F_IN = 128
HID = 32
PERIODS = 12

N_PAD = 10240
CT = 6
COLT = 128
DEGW = 128
CHUNK = 128
NSUB = 16
NBUF = 2
NROWS_B = 157
EP_B = NSUB * NROWS_B * CHUNK
NBUF_A = 4
NROWS_A = 80
EP_A = 32 * NROWS_A * CHUNK
SLICE = N_PAD // NSUB

_mesh = plsc.VectorSubcoreMesh(core_axis_name="c", subcore_axis_name="s")


@functools.partial(
    pl.kernel,
    out_type=jax.ShapeDtypeStruct((2 * N_PAD, DEGW), jnp.float32),
    mesh=_mesh,
    scratch_types=[
        pltpu.VMEM((NROWS_A, CHUNK), jnp.int32),
        pltpu.VMEM((CHUNK, DEGW), jnp.float32),
        pltpu.VMEM_SHARED((N_PAD, DEGW), jnp.float32),
    ],
)
def _deg_kernel(didx_hbm, ones_hbm, zeros_hbm, out_hbm, didx_v, ones_v, accum):
    c = lax.axis_index("c")
    s = lax.axis_index("s")
    w = c * NSUB + s
    pltpu.sync_copy(ones_hbm, ones_v)
    pltpu.sync_copy(didx_hbm.at[pl.ds(w * NROWS_A, NROWS_A)], didx_v)
    pltpu.sync_copy(zeros_hbm.at[pl.ds(s * SLICE, SLICE)],
                    accum.at[pl.ds(s * SLICE, SLICE)])
    plsc.subcore_barrier()

    def body(j, carry):
        pltpu.sync_copy(ones_v, accum.at[didx_v.at[j]], add=True)
        return carry

    lax.fori_loop(0, NROWS_A, body, 0)
    plsc.subcore_barrier()
    pltpu.sync_copy(accum.at[pl.ds(s * SLICE, SLICE)],
                    out_hbm.at[pl.ds(c * N_PAD + s * SLICE, SLICE)])


@functools.partial(
    pl.kernel,
    out_type=jax.ShapeDtypeStruct((CT * N_PAD, COLT), jnp.float32),
    mesh=_mesh,
    scratch_types=[
        pltpu.VMEM((CHUNK,), jnp.int32),
        pltpu.VMEM((CHUNK,), jnp.int32),
        pltpu.VMEM((CHUNK, COLT), jnp.float32),
        pltpu.VMEM_SHARED((N_PAD, COLT), jnp.float32),
        pltpu.SemaphoreType.DMA,
    ],
)
def _prop_kernel(sidx_hbm, didx_hbm, ghat_hbm, zeros_hbm, out_hbm,
                 sidx_v, didx_v, rows_v, accum, sem):
    c = lax.axis_index("c")
    s = lax.axis_index("s")
    for k in range(CT // 2):
        t = c * (CT // 2) + k
        pltpu.sync_copy(zeros_hbm.at[pl.ds(s * SLICE, SLICE)],
                        accum.at[pl.ds(s * SLICE, SLICE)])
        plsc.subcore_barrier()

        def body(j, carry):
            row = s * NROWS_B + j
            pltpu.sync_copy(sidx_hbm.at[t * (NSUB * NROWS_B) + row], sidx_v)
            pltpu.sync_copy(didx_hbm.at[row], didx_v)
            pltpu.async_copy(ghat_hbm.at[sidx_v], rows_v, sem).wait()
            pltpu.sync_copy(rows_v, accum.at[didx_v], add=True)
            return carry

        lax.fori_loop(0, NROWS_B, body, 0)
        plsc.subcore_barrier()
        pltpu.sync_copy(accum.at[pl.ds(s * SLICE, SLICE)],
                        out_hbm.at[pl.ds(t * N_PAD + s * SLICE, SLICE)])


_BN = 1280


def _prep_body(x_ref, d0_ref, d1_ref, wz_ref, lzw_ref, wh_ref, lhw_ref, out_ref):
    wzp = jnp.dot(wz_ref[...], lzw_ref[:HID, :], preferred_element_type=jnp.float32)
    whp = jnp.dot(wh_ref[...], lhw_ref[:HID, :], preferred_element_type=jnp.float32)
    wcat = jnp.concatenate([wzp, whp], axis=1)
    deg = d0_ref[:, 0] + d1_ref[:, 0] + 1.0
    dinv = lax.rsqrt(deg)
    for p in range(PERIODS):
        g = jnp.dot(x_ref[p], wcat, preferred_element_type=jnp.float32)
        t, half = p // 2, p % 2
        out_ref[t, :, half * 64:(half + 1) * 64] = g * dinv[:, None]


def _post_body(c0_ref, gh_ref, d0_ref, d1_ref, att_ref, bz_ref, lzw_ref,
               lzb_ref, bh_ref, lhw_ref, lhb_ref, gam_ref, bet_ref,
               wout_ref, bout_ref, out_ref):
    probs = jax.nn.softmax(att_ref[...], axis=-1)
    bzp = jnp.dot(bz_ref[...], lzw_ref[:HID, :],
                  preferred_element_type=jnp.float32) + lzb_ref[...]
    bhp = jnp.dot(bh_ref[...], lhw_ref[:HID, :],
                  preferred_element_type=jnp.float32) + lhb_ref[...]
    deg = d0_ref[:, 0] + d1_ref[:, 0] + 1.0
    dinv = lax.rsqrt(deg)[:, None]
    acc = jnp.zeros((c0_ref.shape[1], HID), jnp.float32)
    for p in range(PERIODS):
        t, half = p // 2, p % 2
        a = half * 64
        cz = (c0_ref[t, :, a:a + HID] + gh_ref[t, :, a:a + HID]) * dinv + bzp
        ch = (c0_ref[t, :, a + HID:a + 64] + gh_ref[t, :, a + HID:a + 64]) * dinv + bhp
        hp = (1.0 - jax.nn.sigmoid(cz)) * jnp.tanh(ch)
        acc = acc + probs[0:1, p:p + 1] * hp
    h = jnp.where(acc > 0, acc, 0.01 * acc)
    h = (h * (1.0 / math.sqrt(1.0 + 1e-5))) * gam_ref[...] + bet_ref[...]
    out_ref[...] = jnp.dot(h, wout_ref[...],
                           preferred_element_type=jnp.float32) + bout_ref[...]


def kernel(x, edge_index, attention, Wz, bz, LzW, Lzb, Wr, br, LrW, Lrb,
           Wh, bh, LhW, Lhb, gamma, beta, Wout, bout):
    del Wr, br, LrW, Lrb
    f32 = jnp.float32
    s = edge_index[0]
    d = edge_index[1]

    d_a = jnp.concatenate([d, jnp.full((EP_A - E,), N, jnp.int32)])
    didx_a = d_a.reshape(32 * NROWS_A, CHUNK)
    s_b = jnp.concatenate([s, jnp.zeros((EP_B - E,), jnp.int32)])
    d_b = jnp.concatenate([d, jnp.full((EP_B - E,), N, jnp.int32)])
    didx_b = d_b.reshape(NSUB * NROWS_B, CHUNK)
    offs = (jnp.arange(CT, dtype=jnp.int32) * N_PAD)[:, None, None]
    sidx_b = (s_b.reshape(NSUB * NROWS_B, CHUNK)[None] + offs
              ).reshape(CT * NSUB * NROWS_B, CHUNK)

    ones128 = jnp.ones((CHUNK, DEGW), f32)
    zeros128 = jnp.zeros((N_PAD, DEGW), f32)

    xT = jnp.pad(x.transpose(2, 0, 1), ((0, 0), (0, N_PAD - N), (0, 0)))

    degp = _deg_kernel(didx_a, ones128, zeros128)
    deg0 = degp[:N_PAD]
    deg1 = degp[N_PAD:]

    nb = N_PAD // _BN
    ghat = pl.pallas_call(
        _prep_body,
        grid=(nb,),
        in_specs=[
            pl.BlockSpec((PERIODS, _BN, F_IN), lambda i: (0, i, 0)),
            pl.BlockSpec((_BN, DEGW), lambda i: (i, 0)),
            pl.BlockSpec((_BN, DEGW), lambda i: (i, 0)),
            pl.BlockSpec((F_IN, HID), lambda i: (0, 0)),
            pl.BlockSpec((2 * HID, HID), lambda i: (0, 0)),
            pl.BlockSpec((F_IN, HID), lambda i: (0, 0)),
            pl.BlockSpec((2 * HID, HID), lambda i: (0, 0)),
        ],
        out_specs=pl.BlockSpec((CT, _BN, COLT), lambda i: (0, i, 0)),
        out_shape=jax.ShapeDtypeStruct((CT, N_PAD, COLT), f32),
    )(xT, deg0, deg1, Wz, LzW, Wh, LhW)

    c0 = _prop_kernel(sidx_b, didx_b, ghat.reshape(CT * N_PAD, COLT), zeros128)
    c0 = c0.reshape(CT, N_PAD, COLT)

    out = pl.pallas_call(
        _post_body,
        grid=(nb,),
        in_specs=[
            pl.BlockSpec((CT, _BN, COLT), lambda i: (0, i, 0)),
            pl.BlockSpec((CT, _BN, COLT), lambda i: (0, i, 0)),
            pl.BlockSpec((_BN, DEGW), lambda i: (i, 0)),
            pl.BlockSpec((_BN, DEGW), lambda i: (i, 0)),
            pl.BlockSpec((1, PERIODS), lambda i: (0, 0)),
            pl.BlockSpec((1, HID), lambda i: (0, 0)),
            pl.BlockSpec((2 * HID, HID), lambda i: (0, 0)),
            pl.BlockSpec((1, HID), lambda i: (0, 0)),
            pl.BlockSpec((1, HID), lambda i: (0, 0)),
            pl.BlockSpec((2 * HID, HID), lambda i: (0, 0)),
            pl.BlockSpec((1, HID), lambda i: (0, 0)),
            pl.BlockSpec((1, HID), lambda i: (0, 0)),
            pl.BlockSpec((1, HID), lambda i: (0, 0)),
            pl.BlockSpec((HID, PERIODS), lambda i: (0, 0)),
            pl.BlockSpec((1, PERIODS), lambda i: (0, 0)),
        ],
        out_specs=pl.BlockSpec((_BN, PERIODS), lambda i: (i, 0)),
        out_shape=jax.ShapeDtypeStruct((N_PAD, PERIODS), f32),
    )(c0, ghat, deg0, deg1, attention[None, :], bz[None, :], LzW,
      Lzb[None, :], bh[None, :], LhW, Lhb[None, :], gamma[None, :],
      beta[None, :], Wout, bout[None, :])
    return out[:N]
